# baseline (speedup 1.0000x reference)
import functools

import jax
import jax.numpy as jnp
from jax import lax
from jax.experimental import pallas as pl
from jax.experimental.pallas import tpu as pltpu
from jax.experimental.pallas import tpu_sc as plsc

B = 4096
S = 200
D = 64
NUM_CLASSES = 128

NC = 2
NS = 16
NW = NC * NS
B_PER_W = B // NW
SPLIT_A = 104
SPLIT_B = S - SPLIT_A
G_PER_W = 2 * B_PER_W
NBUF = 12
MM_BLK = 2
INV_S = 1.0 / S

_LENS = tuple(SPLIT_A if k % 2 == 0 else SPLIT_B for k in range(NBUF))


def _sc_body(x_hbm, table_hbm, w_hbm, b_hbm, out_hbm, *refs):
    ibs = refs[:NBUF]
    bufs = refs[NBUF:2 * NBUF]
    pooled_v, w_v, b_v, out_v = refs[2 * NBUF:2 * NBUF + 4]
    sems = refs[2 * NBUF + 4:2 * NBUF + 4 + NBUF]
    isems = refs[2 * NBUF + 4 + NBUF:2 * NBUF + 4 + 2 * NBUF]
    wsem = refs[-1]

    wid = lax.axis_index("s") * NC + lax.axis_index("c")
    flat_base = wid * B_PER_W * S

    pltpu.async_copy(w_hbm, w_v, wsem)
    pltpu.sync_copy(b_hbm, b_v)
    pltpu.make_async_copy(w_hbm, w_v, wsem).wait()

    def fetch_idx(g, k):
        off = flat_base + (g // 2) * S + (k % 2) * SPLIT_A
        pltpu.async_copy(x_hbm.at[pl.ds(off, _LENS[k])], ibs[k], isems[k])

    def fire(k):
        pltpu.make_async_copy(x_hbm.at[pl.ds(0, _LENS[k])], ibs[k],
                              isems[k]).wait()
        pltpu.async_copy(table_hbm.at[ibs[k]], bufs[k], sems[k])

    def wait(k):
        pltpu.make_async_copy(table_hbm.at[ibs[k]], bufs[k], sems[k]).wait()

    for k in range(NBUF):
        fetch_idx(k, k)
    for k in range(NBUF):
        fire(k)

    def accumulate(buf, n_rows):
        def body(i, acc):
            a0, a1, a2, a3 = acc
            for u in range(4):
                r = i * 4 + u
                a0 = a0 + buf[r, pl.ds(0, 16)]
                a1 = a1 + buf[r, pl.ds(16, 16)]
                a2 = a2 + buf[r, pl.ds(32, 16)]
                a3 = a3 + buf[r, pl.ds(48, 16)]
            return a0, a1, a2, a3
        zero = jnp.zeros((16,), jnp.float32)
        return lax.fori_loop(0, n_rows // 4, body, (zero, zero, zero, zero))

    def matmul_block(i0):
        binit = tuple(b_v[pl.ds(c * 16, 16)] for c in range(8))

        def body(dblk, acc):
            acc = list(acc)
            pv = [pooled_v[i0 + t, pl.ds(dblk * 16, 16)]
                  for t in range(MM_BLK)]
            for lane in range(16):
                d = dblk * 16 + lane
                w_row = tuple(w_v[d, pl.ds(c * 16, 16)] for c in range(8))
                for t in range(MM_BLK):
                    p = pv[t][lane]
                    for c in range(8):
                        acc[t * 8 + c] = acc[t * 8 + c] + p * w_row[c]
            return tuple(acc)

        acc = lax.fori_loop(0, D // 16, body, binit * MM_BLK)
        for t in range(MM_BLK):
            for c in range(8):
                out_v[i0 + t, pl.ds(c * 16, 16)] = acc[t * 8 + c]

    def outer(j, carry):
        g_base = j * NBUF
        for k in range(NBUF):
            g = g_base + k
            wait(k)

            @pl.when(j < (G_PER_W // NBUF) - 1)
            def _():
                fetch_idx(g + NBUF, k)

            a0, a1, a2, a3 = accumulate(bufs[k], _LENS[k])
            item = (g_base + k) // 2
            if k % 2 == 0:
                pooled_v[item, pl.ds(0, 16)] = a0
                pooled_v[item, pl.ds(16, 16)] = a1
                pooled_v[item, pl.ds(32, 16)] = a2
                pooled_v[item, pl.ds(48, 16)] = a3
            else:
                pooled_v[item, pl.ds(0, 16)] = (
                    pooled_v[item, pl.ds(0, 16)] + a0) * INV_S
                pooled_v[item, pl.ds(16, 16)] = (
                    pooled_v[item, pl.ds(16, 16)] + a1) * INV_S
                pooled_v[item, pl.ds(32, 16)] = (
                    pooled_v[item, pl.ds(32, 16)] + a2) * INV_S
                pooled_v[item, pl.ds(48, 16)] = (
                    pooled_v[item, pl.ds(48, 16)] + a3) * INV_S

            @pl.when(j < (G_PER_W // NBUF) - 1)
            def _():
                fire(k)

            if k % 4 == 3:
                matmul_block(j * (NBUF // 2) + (k - 3) // 2)
        return carry

    lax.fori_loop(0, G_PER_W // NBUF, outer, 0)

    pltpu.sync_copy(out_v, out_hbm.at[pl.ds(wid * B_PER_W, B_PER_W)])


@functools.lru_cache(maxsize=None)
def _make_sc_kernel():
    return pl.kernel(
        _sc_body,
        out_type=jax.ShapeDtypeStruct((B, NUM_CLASSES), jnp.float32),
        mesh=plsc.VectorSubcoreMesh(core_axis_name="c", subcore_axis_name="s",
                                    num_cores=NC, num_subcores=NS),
        compiler_params=pltpu.CompilerParams(use_tc_tiling_on_sc=False),
        scratch_types=(
            [pltpu.VMEM((_LENS[k],), jnp.int32) for k in range(NBUF)]
            + [pltpu.VMEM((_LENS[k], D), jnp.float32) for k in range(NBUF)]
            + [pltpu.VMEM((B_PER_W, D), jnp.float32),
               pltpu.VMEM((D, NUM_CLASSES), jnp.float32),
               pltpu.VMEM((NUM_CLASSES,), jnp.float32),
               pltpu.VMEM((B_PER_W, NUM_CLASSES), jnp.float32)]
            + [pltpu.SemaphoreType.DMA] * (2 * NBUF + 1)
        ),
    )


@jax.jit
def kernel(x, table, W, b):
    xf = x.astype(jnp.int32).reshape(B * S)
    return _make_sc_kernel()(xf, table, W, b)


# speedup vs baseline: 1.4885x; 1.4885x over previous
import functools

import jax
import jax.numpy as jnp
from jax import lax
from jax.experimental import pallas as pl
from jax.experimental.pallas import tpu as pltpu
from jax.experimental.pallas import tpu_sc as plsc

B = 4096
S = 200
D = 64
NUM_CLASSES = 128

NC = 2
NS = 16
NW = NC * NS
B_PER_W = B // NW
SPLIT_A = 104
SPLIT_B = S - SPLIT_A
G_PER_W = 2 * B_PER_W
NBUF = 8
INV_S = 1.0 / S

_LENS = tuple(SPLIT_A if k % 2 == 0 else SPLIT_B for k in range(NBUF))


def _sc_pool_body(x_hbm, table_hbm, out_hbm, *refs):
    ibs = refs[:NBUF]
    bufs = refs[NBUF:2 * NBUF]
    out_v = refs[2 * NBUF]
    sems = refs[2 * NBUF + 1:2 * NBUF + 1 + NBUF]
    isems = refs[2 * NBUF + 1 + NBUF:]

    wid = lax.axis_index("s") * NC + lax.axis_index("c")
    flat_base = wid * B_PER_W * S

    def fetch_idx(g, k):
        off = flat_base + (g // 2) * S + (k % 2) * SPLIT_A
        pltpu.async_copy(x_hbm.at[pl.ds(off, _LENS[k])], ibs[k], isems[k])

    def fire(k):
        pltpu.make_async_copy(x_hbm.at[pl.ds(0, _LENS[k])], ibs[k],
                              isems[k]).wait()
        pltpu.async_copy(table_hbm.at[ibs[k]], bufs[k], sems[k])

    def wait(k):
        pltpu.make_async_copy(table_hbm.at[ibs[k]], bufs[k], sems[k]).wait()

    for k in range(NBUF):
        fetch_idx(k, k)
    for k in range(NBUF):
        fire(k)

    def accumulate(buf, n_rows):
        def body(i, acc):
            a0, a1, a2, a3 = acc
            for u in range(4):
                r = i * 4 + u
                a0 = a0 + buf[r, pl.ds(0, 16)]
                a1 = a1 + buf[r, pl.ds(16, 16)]
                a2 = a2 + buf[r, pl.ds(32, 16)]
                a3 = a3 + buf[r, pl.ds(48, 16)]
            return a0, a1, a2, a3
        zero = jnp.zeros((16,), jnp.float32)
        return lax.fori_loop(0, n_rows // 4, body, (zero, zero, zero, zero))

    def outer(j, carry):
        g_base = j * NBUF
        for k in range(NBUF):
            g = g_base + k
            wait(k)

            @pl.when(j < (G_PER_W // NBUF) - 1)
            def _():
                fetch_idx(g + NBUF, k)

            a0, a1, a2, a3 = accumulate(bufs[k], _LENS[k])
            item = (g_base + k) // 2
            if k % 2 == 0:
                out_v[item, pl.ds(0, 16)] = a0
                out_v[item, pl.ds(16, 16)] = a1
                out_v[item, pl.ds(32, 16)] = a2
                out_v[item, pl.ds(48, 16)] = a3
            else:
                out_v[item, pl.ds(0, 16)] = (
                    out_v[item, pl.ds(0, 16)] + a0) * INV_S
                out_v[item, pl.ds(16, 16)] = (
                    out_v[item, pl.ds(16, 16)] + a1) * INV_S
                out_v[item, pl.ds(32, 16)] = (
                    out_v[item, pl.ds(32, 16)] + a2) * INV_S
                out_v[item, pl.ds(48, 16)] = (
                    out_v[item, pl.ds(48, 16)] + a3) * INV_S

            @pl.when(j < (G_PER_W // NBUF) - 1)
            def _():
                fire(k)
        return carry

    lax.fori_loop(0, G_PER_W // NBUF, outer, 0)

    pltpu.sync_copy(out_v, out_hbm.at[pl.ds(wid * B_PER_W, B_PER_W)])


@functools.lru_cache(maxsize=None)
def _make_sc_pool():
    return pl.kernel(
        _sc_pool_body,
        out_type=jax.ShapeDtypeStruct((B, NUM_CLASSES), jnp.float32),
        mesh=plsc.VectorSubcoreMesh(core_axis_name="c", subcore_axis_name="s",
                                    num_cores=NC, num_subcores=NS),
        compiler_params=pltpu.CompilerParams(use_tc_tiling_on_sc=False),
        scratch_types=(
            [pltpu.VMEM((_LENS[k],), jnp.int32) for k in range(NBUF)]
            + [pltpu.VMEM((_LENS[k], D), jnp.float32) for k in range(NBUF)]
            + [pltpu.VMEM((B_PER_W, NUM_CLASSES), jnp.float32)]
            + [pltpu.SemaphoreType.DMA] * (2 * NBUF)
        ),
    )


def _mm_body(p_ref, w_ref, b_ref, o_ref):
    o_ref[...] = jnp.dot(p_ref[:, :D], w_ref[...],
                         preferred_element_type=jnp.float32) + b_ref[...]


def _tc_matmul(pooled_pad, W, b):
    blk = 512
    return pl.pallas_call(
        _mm_body,
        grid=(B // blk,),
        in_specs=[
            pl.BlockSpec((blk, NUM_CLASSES), lambda i: (i, 0)),
            pl.BlockSpec((D, NUM_CLASSES), lambda i: (0, 0)),
            pl.BlockSpec((NUM_CLASSES,), lambda i: (0,)),
        ],
        out_specs=pl.BlockSpec((blk, NUM_CLASSES), lambda i: (i, 0)),
        out_shape=jax.ShapeDtypeStruct((B, NUM_CLASSES), jnp.float32),
    )(pooled_pad, W, b)


@jax.jit
def kernel(x, table, W, b):
    xf = x.astype(jnp.int32).reshape(B * S)
    pooled_pad = _make_sc_pool()(xf, table)
    return _tc_matmul(pooled_pad, W, b)
